# Initial kernel scaffold; baseline (speedup 1.0000x reference)
import jax
import jax.numpy as jnp
from jax.experimental import pallas as pl


def kernel(x, adj, gc_W, gc_b, root_emb, in_proj_w, in_proj_b, out_proj_w, out_proj_b, ln1_g, ln1_b, ln2_g, ln2_b, gate_w, e_w1, e_b1, e_w2, e_b2):
    raise NotImplementedError("write your pallas kernel here")


# speedup vs baseline: 1.9224x; 1.9224x over previous
import functools
import jax
import jax.numpy as jnp
from jax import lax
from jax.experimental import pallas as pl
from jax.experimental.pallas import tpu as pltpu

N, D = 2048, 768
E, DFF, H = 8, 1024, 12
DH = D // H
BLK = 256
NBLK = N // BLK
_EPS = 1e-5

_DNT = (((1,), (1,)), ((), ()))


def _ln(x, g, b):
    m = jnp.mean(x, axis=-1, keepdims=True)
    v = jnp.mean((x - m) ** 2, axis=-1, keepdims=True)
    return (x - m) * lax.rsqrt(v + _EPS) * g + b


def _gc_qkv_kernel(adj_ref, xfull_ref, xblk_ref, gcw_ref, gcb_ref, root_ref,
                   inw_ref, inb_ref, x1_ref, qkv_ref):
    g = jnp.dot(adj_ref[...], xfull_ref[...], preferred_element_type=jnp.float32)
    g = lax.dot_general(g, gcw_ref[...], _DNT, preferred_element_type=jnp.float32)
    g = g + gcb_ref[...] + root_ref[...]
    x1 = xblk_ref[...] + g
    x1_ref[...] = x1
    qkv_ref[...] = lax.dot_general(x1, inw_ref[...], _DNT,
                                   preferred_element_type=jnp.float32) + inb_ref[...]


def _attn_kernel(qkvblk_ref, qkvfull_ref, out_ref):
    scale = 1.0 / (DH ** 0.5)
    for h in range(H):
        q = qkvblk_ref[:, h * DH:(h + 1) * DH]
        k = qkvfull_ref[:, D + h * DH:D + (h + 1) * DH]
        v = qkvfull_ref[:, 2 * D + h * DH:2 * D + (h + 1) * DH]
        s = lax.dot_general(q, k, _DNT, preferred_element_type=jnp.float32) * scale
        m = jnp.max(s, axis=-1, keepdims=True)
        p = jnp.exp(s - m)
        p = p / jnp.sum(p, axis=-1, keepdims=True)
        out_ref[:, h * DH:(h + 1) * DH] = jnp.dot(p, v, preferred_element_type=jnp.float32)


def _post_attn_kernel(attn_ref, x1_ref, outw_ref, outb_ref, ln1g_ref, ln1b_ref,
                      gatew_ref, x2_ref, comb_ref):
    a = lax.dot_general(attn_ref[...], outw_ref[...], _DNT,
                        preferred_element_type=jnp.float32) + outb_ref[...]
    pre = x1_ref[...] + a
    x2 = _ln(pre, ln1g_ref[...], ln1b_ref[...])
    x2_ref[...] = x2
    gl = lax.dot_general(x2, gatew_ref[...], _DNT, preferred_element_type=jnp.float32)
    l1 = jnp.max(gl, axis=-1, keepdims=True)
    iota = lax.broadcasted_iota(jnp.int32, gl.shape, 1)
    a1 = jnp.min(jnp.where(gl == l1, iota, E), axis=-1, keepdims=True)
    gl2 = jnp.where(iota == a1, -jnp.inf, gl)
    l2 = jnp.max(gl2, axis=-1, keepdims=True)
    a2 = jnp.min(jnp.where(gl2 == l2, iota, E), axis=-1, keepdims=True)
    w1 = 1.0 / (1.0 + jnp.exp(l2 - l1))
    w2 = 1.0 - w1
    comb_ref[...] = jnp.where(iota == a1, w1, 0.0) + jnp.where(iota == a2, w2, 0.0)


def _moe_dense_kernel(x2_ref, comb_ref, w1_ref, b1_ref, w2_ref, b2_ref,
                      ln2g_ref, ln2b_ref, out_ref, acc_ref):
    e = pl.program_id(1)

    @pl.when(e == 0)
    def _():
        acc_ref[...] = jnp.zeros_like(acc_ref)

    x2 = x2_ref[...]
    h = lax.dot_general(x2, w1_ref[0], _DNT, preferred_element_type=jnp.float32) + b1_ref[0]
    h = 0.5 * h * (1.0 + lax.erf(h * (2.0 ** -0.5)))
    y = lax.dot_general(h, w2_ref[0], _DNT, preferred_element_type=jnp.float32) + b2_ref[0]
    c = comb_ref[...]
    iota = lax.broadcasted_iota(jnp.int32, c.shape, 1)
    ce = jnp.sum(jnp.where(iota == e, c, 0.0), axis=-1, keepdims=True)
    acc_ref[...] += ce * y

    @pl.when(e == E - 1)
    def _():
        out_ref[...] = _ln(x2 + acc_ref[...], ln2g_ref[...], ln2b_ref[...])


def kernel(x, adj, gc_W, gc_b, root_emb, in_proj_w, in_proj_b, out_proj_w,
           out_proj_b, ln1_g, ln1_b, ln2_g, ln2_b, gate_w, e_w1, e_b1, e_w2, e_b2):
    xf = x.reshape(N, D)
    adjf = adj.reshape(N, N)
    row = lambda a: a.reshape(1, -1)

    x1, qkv = pl.pallas_call(
        _gc_qkv_kernel,
        grid=(NBLK,),
        in_specs=[
            pl.BlockSpec((BLK, N), lambda i: (i, 0)),
            pl.BlockSpec((N, D), lambda i: (0, 0)),
            pl.BlockSpec((BLK, D), lambda i: (i, 0)),
            pl.BlockSpec((D, D), lambda i: (0, 0)),
            pl.BlockSpec((1, D), lambda i: (0, 0)),
            pl.BlockSpec((1, D), lambda i: (0, 0)),
            pl.BlockSpec((3 * D, D), lambda i: (0, 0)),
            pl.BlockSpec((1, 3 * D), lambda i: (0, 0)),
        ],
        out_specs=[pl.BlockSpec((BLK, D), lambda i: (i, 0)),
                   pl.BlockSpec((BLK, 3 * D), lambda i: (i, 0))],
        out_shape=[jax.ShapeDtypeStruct((N, D), jnp.float32),
                   jax.ShapeDtypeStruct((N, 3 * D), jnp.float32)],
    )(adjf, xf, xf, gc_W, row(gc_b), root_emb, in_proj_w, row(in_proj_b))

    attn = pl.pallas_call(
        _attn_kernel,
        grid=(NBLK,),
        in_specs=[
            pl.BlockSpec((BLK, 3 * D), lambda i: (i, 0)),
            pl.BlockSpec((N, 3 * D), lambda i: (0, 0)),
        ],
        out_specs=pl.BlockSpec((BLK, D), lambda i: (i, 0)),
        out_shape=jax.ShapeDtypeStruct((N, D), jnp.float32),
    )(qkv, qkv)

    x2, comb = pl.pallas_call(
        _post_attn_kernel,
        grid=(NBLK,),
        in_specs=[
            pl.BlockSpec((BLK, D), lambda i: (i, 0)),
            pl.BlockSpec((BLK, D), lambda i: (i, 0)),
            pl.BlockSpec((D, D), lambda i: (0, 0)),
            pl.BlockSpec((1, D), lambda i: (0, 0)),
            pl.BlockSpec((1, D), lambda i: (0, 0)),
            pl.BlockSpec((1, D), lambda i: (0, 0)),
            pl.BlockSpec((E, D), lambda i: (0, 0)),
        ],
        out_specs=[pl.BlockSpec((BLK, D), lambda i: (i, 0)),
                   pl.BlockSpec((BLK, E), lambda i: (i, 0))],
        out_shape=[jax.ShapeDtypeStruct((N, D), jnp.float32),
                   jax.ShapeDtypeStruct((N, E), jnp.float32)],
    )(attn, x1, out_proj_w, row(out_proj_b), row(ln1_g), row(ln1_b), gate_w)

    out = pl.pallas_call(
        _moe_dense_kernel,
        grid=(NBLK, E),
        in_specs=[
            pl.BlockSpec((BLK, D), lambda i, e: (i, 0)),
            pl.BlockSpec((BLK, E), lambda i, e: (i, 0)),
            pl.BlockSpec((1, DFF, D), lambda i, e: (e, 0, 0)),
            pl.BlockSpec((1, 1, DFF), lambda i, e: (e, 0, 0)),
            pl.BlockSpec((1, D, DFF), lambda i, e: (e, 0, 0)),
            pl.BlockSpec((1, 1, D), lambda i, e: (e, 0, 0)),
            pl.BlockSpec((1, D), lambda i, e: (0, 0)),
            pl.BlockSpec((1, D), lambda i, e: (0, 0)),
        ],
        out_specs=pl.BlockSpec((BLK, D), lambda i, e: (i, 0)),
        out_shape=jax.ShapeDtypeStruct((N, D), jnp.float32),
        scratch_shapes=[pltpu.VMEM((BLK, D), jnp.float32)],
    )(x2, comb, e_w1, e_b1.reshape(E, 1, DFF), e_w2, e_b2.reshape(E, 1, D),
      row(ln2_g), row(ln2_b))

    return out.reshape(x.shape)
